# baseline (speedup 1.0000x reference)
import functools

import jax
import jax.numpy as jnp
from jax import lax
from jax.experimental import pallas as pl
from jax.experimental.pallas import tpu as pltpu
from jax.experimental.pallas import tpu_sc as plsc

NC = 2
NS = 16
NW = NC * NS
NBUF = 16
LOOKAHEAD = 8
BCH = 32
L = 16


@jax.jit
def _gather_sc(ids, weight):
    B, F = ids.shape
    D = weight.shape[1]
    FD = F * D
    rows_per_w = B // NW
    n_win = FD // L
    mesh = plsc.VectorSubcoreMesh(core_axis_name="c", subcore_axis_name="s")

    @functools.partial(
        pl.kernel,
        mesh=mesh,
        compiler_params=pltpu.CompilerParams(
            use_tc_tiling_on_sc=False, needs_layout_passes=False
        ),
        out_type=jax.ShapeDtypeStruct((FD, B), jnp.float32),
        scratch_types=[
            pltpu.VMEM((rows_per_w, F), jnp.int32),
            pltpu.VMEM((NBUF, F, D), jnp.float32),
            pltpu.VMEM((2 * FD, BCH), jnp.float32),
            pltpu.SemaphoreType.DMA,
            pltpu.SemaphoreType.DMA,
        ],
    )
    def k(idx_hbm, table_hbm, out_hbm, idx_v, rows_v, blk_v, gsem, bsem):
        wid = lax.axis_index("s") * NC + lax.axis_index("c")
        row0 = wid * rows_per_w
        pltpu.sync_copy(idx_hbm.at[pl.ds(row0, rows_per_w)], idx_v)

        def fire_gather(r):
            pltpu.async_copy(
                table_hbm.at[idx_v.at[r]], rows_v.at[r % NBUF], gsem
            )

        def wait_sem(sem, src, dst):
            pltpu.make_async_copy(src, dst, sem).wait()

        for r in range(LOOKAHEAD):
            fire_gather(r)

        iota = lax.broadcasted_iota(jnp.int32, (L,), 0)

        def body(r, carry):
            blk = r // BCH
            j = r % BCH
            base = (blk % 2) * FD

            @pl.when(j == 0)
            def _():
                @pl.when(blk >= 2)
                def _():
                    wait_sem(
                        bsem,
                        blk_v.at[pl.ds(0, FD)],
                        out_hbm.at[:, pl.ds(0, BCH)],
                    )

            @pl.when(r + LOOKAHEAD < rows_per_w)
            def _():
                fire_gather(r + LOOKAHEAD)

            wait_sem(gsem, out_hbm.at[pl.ds(0, F), pl.ds(0, D)], rows_v.at[0])

            rb = r % NBUF
            b_idx = jnp.full((L,), j, jnp.int32)
            for w in range(n_win):
                f = w // (D // L)
                h = w % (D // L)
                v = rows_v[rb, f, pl.ds(h * L, L)]
                fd_idx = iota + (base + w * L)
                plsc.store_scatter(blk_v, [fd_idx, b_idx], v)

            @pl.when(j == BCH - 1)
            def _():
                pltpu.async_copy(
                    blk_v.at[pl.ds(base, FD)],
                    out_hbm.at[:, pl.ds(row0 + blk * BCH, BCH)],
                    bsem,
                )

            return carry

        lax.fori_loop(0, rows_per_w, body, 0)
        for _ in range(2):
            wait_sem(bsem, blk_v.at[pl.ds(0, FD)], out_hbm.at[:, pl.ds(0, BCH)])

    out_fm = k(ids, weight)
    return jnp.transpose(out_fm.reshape(F, D, B), (2, 0, 1))


def kernel(token_ids, weight):
    return _gather_sc(token_ids.astype(jnp.int32), weight)
